# baseline (speedup 1.0000x reference)
import functools

import jax
import jax.numpy as jnp
from jax import lax
from jax.experimental import pallas as pl
from jax.experimental.pallas import tpu as pltpu
from jax.experimental.pallas import tpu_sc as plsc

BATCH = 16384
FEAT = 128
NUM_CORES = 2
NUM_SUBCORES = 16
NUM_WORKERS = NUM_CORES * NUM_SUBCORES
ROWS_PER_WORKER = BATCH // NUM_WORKERS
CHUNK = 128
NCHUNK = ROWS_PER_WORKER // CHUNK
LANES = 16

_mesh = plsc.VectorSubcoreMesh(core_axis_name="c", subcore_axis_name="s")


@functools.partial(
    pl.kernel,
    mesh=_mesh,
    out_type=jax.ShapeDtypeStruct((BATCH, FEAT), jnp.float32),
    scratch_types=[
        pltpu.VMEM((ROWS_PER_WORKER,), jnp.int32),
        pltpu.VMEM((2, CHUNK, FEAT), jnp.float32),
        pltpu.VMEM((2, CHUNK, FEAT), jnp.float32),
        pltpu.VMEM((2, CHUNK, FEAT), jnp.float32),
        pltpu.SemaphoreType.DMA,
        pltpu.SemaphoreType.DMA,
        pltpu.SemaphoreType.DMA,
        pltpu.SemaphoreType.DMA,
    ],
)
def _film_sc(x_hbm, ids_hbm, gamma_hbm, beta_hbm, out_hbm,
             idx_v, g_v, b_v, x_v, sem_g, sem_b, sem_x, sem_o):
    wid = lax.axis_index("s") * NUM_CORES + lax.axis_index("c")
    base = wid * ROWS_PER_WORKER

    pltpu.sync_copy(ids_hbm.at[pl.ds(base, ROWS_PER_WORKER)], idx_v)

    def issue(c):
        s = c % 2
        off = base + c * CHUNK
        return (
            pltpu.async_copy(gamma_hbm.at[idx_v.at[pl.ds(c * CHUNK, CHUNK)]], g_v.at[s], sem_g),
            pltpu.async_copy(x_hbm.at[pl.ds(off, CHUNK)], x_v.at[s], sem_x),
        )

    handles = issue(0)
    wb = None
    for c in range(NCHUNK):
        s = c % 2
        if c + 1 < NCHUNK:
            if wb is not None:
                wb.wait()
                wb = None
            nxt = issue(c + 1)
        for h in handles:
            h.wait()

        def row_body(r, carry):
            for j in range(FEAT // LANES):
                sl = pl.ds(j * LANES, LANES)
                g_v[s, r, sl] = g_v[s, r, sl] * x_v[s, r, sl]
            return carry

        lax.fori_loop(0, CHUNK, row_body, 0)
        pltpu.async_copy(beta_hbm.at[idx_v.at[pl.ds(c * CHUNK, CHUNK)]],
                         g_v.at[s], sem_b, add=True).wait()
        if wb is not None:
            wb.wait()
        wb = pltpu.async_copy(g_v.at[s], out_hbm.at[pl.ds(base + c * CHUNK, CHUNK)], sem_o)
        if c + 1 < NCHUNK:
            handles = nxt
    wb.wait()


def kernel(x, domain_ids, gamma, beta):
    return _film_sc(x, domain_ids.astype(jnp.int32), gamma, beta)


# speedup vs baseline: 1.1062x; 1.1062x over previous
import functools

import jax
import jax.numpy as jnp
from jax import lax
from jax.experimental import pallas as pl
from jax.experimental.pallas import tpu as pltpu
from jax.experimental.pallas import tpu_sc as plsc

BATCH = 16384
FEAT = 128
NUM_CORES = 2
NUM_SUBCORES = 16
NUM_WORKERS = NUM_CORES * NUM_SUBCORES
ROWS_PER_WORKER = BATCH // NUM_WORKERS
CHUNK = 128
NCHUNK = ROWS_PER_WORKER // CHUNK
LANES = 16

_mesh = plsc.VectorSubcoreMesh(core_axis_name="c", subcore_axis_name="s")


@functools.partial(
    pl.kernel,
    mesh=_mesh,
    out_type=jax.ShapeDtypeStruct((BATCH, FEAT), jnp.float32),
    scratch_types=[
        pltpu.VMEM((ROWS_PER_WORKER,), jnp.int32),
        pltpu.VMEM((NCHUNK, CHUNK, FEAT), jnp.float32),
        pltpu.VMEM((2, CHUNK, FEAT), jnp.float32),
        pltpu.SemaphoreType.DMA,
        pltpu.SemaphoreType.DMA,
        pltpu.SemaphoreType.DMA,
        pltpu.SemaphoreType.DMA,
    ],
)
def _film_sc(x_hbm, ids_hbm, gamma_hbm, beta_hbm, out_hbm,
             idx_v, g_v, x_v, sem_g, sem_b, sem_x, sem_o):
    wid = lax.axis_index("s") * NUM_CORES + lax.axis_index("c")
    base = wid * ROWS_PER_WORKER

    pltpu.sync_copy(ids_hbm.at[pl.ds(base, ROWS_PER_WORKER)], idx_v)

    def issue_g(c):
        return pltpu.async_copy(
            gamma_hbm.at[idx_v.at[pl.ds(c * CHUNK, CHUNK)]], g_v.at[c], sem_g)

    def issue_x(c):
        return pltpu.async_copy(
            x_hbm.at[pl.ds(base + c * CHUNK, CHUNK)], x_v.at[c % 2], sem_x)

    hg = [None] * NCHUNK
    hx = [None] * NCHUNK
    badd = [None] * NCHUNK
    wb = [None] * NCHUNK
    hg[0] = issue_g(0)
    hx[0] = issue_x(0)
    hg[1] = issue_g(1)
    hx[1] = issue_x(1)

    for c in range(NCHUNK):
        hg[c].wait()
        hx[c].wait()
        if c + 2 < NCHUNK:
            hg[c + 2] = issue_g(c + 2)

        def row_body(r, carry):
            for j in range(FEAT // LANES):
                sl = pl.ds(j * LANES, LANES)
                g_v[c, r, sl] = g_v[c, r, sl] * x_v[c % 2, r, sl]
            return carry

        lax.fori_loop(0, CHUNK, row_body, 0)
        badd[c] = pltpu.async_copy(beta_hbm.at[idx_v.at[pl.ds(c * CHUNK, CHUNK)]],
                                   g_v.at[c], sem_b, add=True)
        if c + 2 < NCHUNK:
            hx[c + 2] = issue_x(c + 2)
        if c >= 1:
            badd[c - 1].wait()
            wb[c - 1] = pltpu.async_copy(
                g_v.at[c - 1], out_hbm.at[pl.ds(base + (c - 1) * CHUNK, CHUNK)], sem_o)

    badd[NCHUNK - 1].wait()
    wb[NCHUNK - 1] = pltpu.async_copy(
        g_v.at[NCHUNK - 1],
        out_hbm.at[pl.ds(base + (NCHUNK - 1) * CHUNK, CHUNK)], sem_o)
    for h in wb:
        h.wait()


def kernel(x, domain_ids, gamma, beta):
    return _film_sc(x, domain_ids.astype(jnp.int32), gamma, beta)
